# baseline (speedup 1.0000x reference)
import functools

import jax
import jax.numpy as jnp
from jax import lax
from jax.experimental import pallas as pl
from jax.experimental.pallas import tpu as pltpu
from jax.experimental.pallas import tpu_sc as plsc

N, D, H, NB = 10000, 128, 128, 32
DEPTH = 6

_info = plsc.get_sparse_core_info()
NC, NS, L = _info.num_cores, _info.num_subcores, _info.num_lanes
NW = NC * NS
NPAD = 10240
G = 64
NP0, NP1 = 10, 0
CHUNK0, CHUNK1 = NP0 * G, NP1 * G
SPLIT = NS * CHUNK0
NSLOT = 4


def _sc_gather_sum(idx_r, msg):
    mesh = plsc.VectorSubcoreMesh(core_axis_name="c", subcore_axis_name="s")

    @functools.partial(
        pl.kernel,
        out_type=jax.ShapeDtypeStruct((NPAD, D), jnp.float32),
        mesh=mesh,
        scratch_types=[
            pltpu.VMEM((NB, NP0, G), jnp.int32),
            pltpu.VMEM((NSLOT, G, D), jnp.float32),
            pltpu.VMEM((1, G), jnp.int32),
            pltpu.VMEM_SHARED((NS * G, D), jnp.float32),
            [pltpu.SemaphoreType.DMA] * NSLOT,
            [pltpu.SemaphoreType.DMA] * NSLOT,
        ],
    )
    def body(idx_hbm, msg_hbm, out_hbm, idx_v, buf_v, lin_v, acc_s,
             gsem, ssem):
        ss = lax.axis_index("s")
        cc = lax.axis_index("c")
        wid = cc * NS + ss
        npass = jnp.where(cc == 0, NP0, NP1)
        obase = jnp.where(cc == 0, ss * CHUNK0, SPLIT + ss * CHUNK1)
        abase = ss * G
        for t in range(G // L):
            lin_v[0, pl.ds(t * L, L)] = (
                jnp.arange(L, dtype=jnp.int32) + abase + t * L
            )
        pltpu.sync_copy(idx_hbm.at[wid], idx_v)

        def g_start(j, p, slot):
            pltpu.async_copy(
                msg_hbm.at[idx_v.at[j, p]], buf_v.at[slot], gsem[slot]
            )

        def g_wait(slot):
            pltpu.make_async_copy(
                msg_hbm.at[idx_v.at[0, 0]], buf_v.at[slot], gsem[slot]
            ).wait()

        def s_start(slot):
            pltpu.async_copy(
                buf_v.at[slot], acc_s.at[lin_v.at[0]], ssem[slot], add=True
            )

        def s_wait(slot):
            pltpu.make_async_copy(
                buf_v.at[slot], acc_s.at[lin_v.at[0]], ssem[slot]
            ).wait()

        def run_pass(p, _):
            for b in range(NSLOT):
                g_start(b, p, b)
            g_wait(0)
            pltpu.async_copy(buf_v.at[0], acc_s.at[pl.ds(abase, G)], ssem[0])
            pltpu.make_async_copy(
                buf_v.at[0], acc_s.at[pl.ds(abase, G)], ssem[0]
            ).wait()
            g_start(NSLOT, p, 0)
            for b in range(1, NSLOT):
                g_wait(b)
                s_start(b)
                s_wait(b)
                g_start(b + NSLOT, p, b)

            def group_step(jj, _):
                for b in range(NSLOT):
                    j = NSLOT * jj + b
                    g_wait(b)
                    s_start(b)
                    s_wait(b)

                    @pl.when(j + NSLOT < NB)
                    def _():
                        g_start(j + NSLOT, p, b)

                return 0

            lax.fori_loop(1, NB // NSLOT, group_step, 0)
            pltpu.sync_copy(
                acc_s.at[pl.ds(abase, G)],
                out_hbm.at[pl.ds(obase + p * G, G)],
            )
            return 0

        @pl.when(cc == 0)
        def _():
            lax.fori_loop(0, npass, run_pass, 0)


    return body(idx_r, msg)


def _tc_matmul_relu(x, wt):
    bm = 1024

    def body(x_ref, w_ref, o_ref):
        o_ref[...] = jnp.maximum(
            jnp.dot(x_ref[...], w_ref[...], preferred_element_type=jnp.float32),
            0.0,
        )

    return pl.pallas_call(
        body,
        grid=(NPAD // bm,),
        in_specs=[
            pl.BlockSpec((bm, D), lambda i: (i, 0)),
            pl.BlockSpec((D, H), lambda i: (0, 0)),
        ],
        out_specs=pl.BlockSpec((bm, H), lambda i: (i, 0)),
        out_shape=jax.ShapeDtypeStruct((NPAD, H), jnp.float32),
    )(x, wt)


def kernel(init_messages, init_attached_features, a2nei, a2attached, W_i, W_h):
    del init_attached_features, a2attached
    idx = jnp.pad(a2nei.astype(jnp.int32), ((0, NPAD - N), (0, 0)))
    idx_t = idx.T
    idx0 = (
        idx_t.reshape(NB, NS, NP0, G).transpose(1, 0, 2, 3)
    )
    idx_r = jnp.concatenate([idx0, jnp.zeros_like(idx0)], axis=0)

    x = jnp.pad(init_messages, ((0, NPAD - N), (0, 0)))
    msg = _tc_matmul_relu(x, W_i.T)
    for _ in range(DEPTH - 1):
        s = _sc_gather_sum(idx_r, msg)
        msg = _tc_matmul_relu(s, W_h.T)
    return msg[:N]


# speedup vs baseline: 2.7926x; 2.7926x over previous
import functools

import jax
import jax.numpy as jnp
from jax import lax
from jax.experimental import pallas as pl
from jax.experimental.pallas import tpu as pltpu
from jax.experimental.pallas import tpu_sc as plsc

N, D, H, NB = 10000, 128, 128, 32
DEPTH = 6

_info = plsc.get_sparse_core_info()
NC, NS, L = _info.num_cores, _info.num_subcores, _info.num_lanes
NW = NC * NS
NPAD = 10240
G = 64
NP0, NP1 = 10, 0
CHUNK0, CHUNK1 = NP0 * G, NP1 * G
SPLIT = NS * CHUNK0
NSLOT = 4


def _sc_gather_sum(idx_r, msg):
    mesh = plsc.VectorSubcoreMesh(core_axis_name="c", subcore_axis_name="s")

    @functools.partial(
        pl.kernel,
        out_type=jax.ShapeDtypeStruct((NPAD, D), jnp.float32),
        mesh=mesh,
        scratch_types=[
            pltpu.VMEM((NB, NP0, G), jnp.int32),
            pltpu.VMEM((NSLOT, G, D), jnp.float32),
            pltpu.VMEM((1, G), jnp.int32),
            pltpu.VMEM_SHARED((NS * G, D), jnp.float32),
            [pltpu.SemaphoreType.DMA] * NSLOT,
            [pltpu.SemaphoreType.DMA] * NSLOT,
        ],
    )
    def body(idx_hbm, msg_hbm, out_hbm, idx_v, buf_v, lin_v, acc_s,
             gsem, ssem):
        ss = lax.axis_index("s")
        cc = lax.axis_index("c")
        wid = cc * NS + ss
        npass = jnp.where(cc == 0, NP0, NP1)
        obase = jnp.where(cc == 0, ss * CHUNK0, SPLIT + ss * CHUNK1)
        abase = ss * G
        for t in range(G // L):
            lin_v[0, pl.ds(t * L, L)] = (
                jnp.arange(L, dtype=jnp.int32) + abase + t * L
            )
        pltpu.sync_copy(idx_hbm.at[wid], idx_v)

        def g_start(j, p, slot):
            pltpu.async_copy(
                msg_hbm.at[idx_v.at[j, p]], buf_v.at[slot], gsem[slot]
            )

        def g_wait(slot):
            pltpu.make_async_copy(
                msg_hbm.at[idx_v.at[0, 0]], buf_v.at[slot], gsem[slot]
            ).wait()

        def s_start(slot):
            pltpu.async_copy(
                buf_v.at[slot], acc_s.at[lin_v.at[0]], ssem[slot], add=True
            )

        def s_wait(slot):
            pltpu.make_async_copy(
                buf_v.at[slot], acc_s.at[lin_v.at[0]], ssem[slot]
            ).wait()

        def run_pass(p, _):
            for b in range(NSLOT):
                g_start(b, p, b)
            g_wait(0)
            pltpu.async_copy(buf_v.at[0], acc_s.at[pl.ds(abase, G)], ssem[0])
            pltpu.make_async_copy(
                buf_v.at[0], acc_s.at[pl.ds(abase, G)], ssem[0]
            ).wait()
            g_start(NSLOT, p, 0)
            for b in range(1, NSLOT):
                g_wait(b)
                s_start(b)
                s_wait(b)
                g_start(b + NSLOT, p, b)

            def group_step(jj, _):
                for b in range(NSLOT):
                    j = NSLOT * jj + b
                    g_wait(b)
                    s_start(b)
                    s_wait(b)

                    @pl.when(j + NSLOT < NB)
                    def _():
                        g_start(j + NSLOT, p, b)

                return 0

            lax.fori_loop(1, NB // NSLOT, group_step, 0)
            pltpu.sync_copy(
                acc_s.at[pl.ds(abase, G)],
                out_hbm.at[pl.ds(obase + p * G, G)],
            )
            return 0

        @pl.when(cc == 0)
        def _():
            lax.fori_loop(0, npass, run_pass, 0)


    return body(idx_r, msg)


def _tc_matmul_relu(x, wt):
    bm = 1024

    def body(x_ref, w_ref, o_ref):
        o_ref[...] = jnp.maximum(
            jnp.dot(x_ref[...], w_ref[...], preferred_element_type=jnp.float32),
            0.0,
        )

    return pl.pallas_call(
        body,
        grid=(NPAD // bm,),
        in_specs=[
            pl.BlockSpec((bm, D), lambda i: (i, 0)),
            pl.BlockSpec((D, H), lambda i: (0, 0)),
        ],
        out_specs=pl.BlockSpec((bm, H), lambda i: (i, 0)),
        out_shape=jax.ShapeDtypeStruct((NPAD, H), jnp.float32),
    )(x, wt)


def kernel(init_messages, init_attached_features, a2nei, a2attached, W_i, W_h):
    del init_attached_features, a2attached
    fill = (
        jnp.arange(NPAD - N, dtype=jnp.int32)[:, None] * NB
        + jnp.arange(NB, dtype=jnp.int32)[None, :]
    ) % N
    idx = jnp.concatenate([a2nei.astype(jnp.int32), fill], axis=0)
    idx_t = idx.T
    idx0 = (
        idx_t.reshape(NB, NS, NP0, G).transpose(1, 0, 2, 3)
    )
    idx_r = jnp.concatenate([idx0, jnp.zeros_like(idx0)], axis=0)

    x = jnp.pad(init_messages, ((0, NPAD - N), (0, 0)))
    msg = _tc_matmul_relu(x, W_i.T)
    for _ in range(DEPTH - 1):
        s = _sc_gather_sum(idx_r, msg)
        msg = _tc_matmul_relu(s, W_h.T)
    return msg[:N]


# speedup vs baseline: 5.0402x; 1.8049x over previous
import functools

import jax
import jax.numpy as jnp
from jax import lax
from jax.experimental import pallas as pl
from jax.experimental.pallas import tpu as pltpu
from jax.experimental.pallas import tpu_sc as plsc

N, D, H, NB = 10000, 128, 128, 32
DEPTH = 6

_info = plsc.get_sparse_core_info()
NC, NS, L = _info.num_cores, _info.num_subcores, _info.num_lanes
NW = NC * NS
NPAD = 10240
G = 64
NP0, NP1 = 5, 5
CHUNK0, CHUNK1 = NP0 * G, NP1 * G
SPLIT = NS * CHUNK0
NSLOT = 4


def _sc_gather_sum(idx_r, msg):
    mesh = plsc.VectorSubcoreMesh(core_axis_name="c", subcore_axis_name="s")

    @functools.partial(
        pl.kernel,
        out_type=jax.ShapeDtypeStruct((NPAD, D), jnp.float32),
        mesh=mesh,
        scratch_types=[
            pltpu.VMEM((NB, NP0, G), jnp.int32),
            pltpu.VMEM((NSLOT, G, D), jnp.float32),
            pltpu.VMEM((1, G), jnp.int32),
            pltpu.VMEM_SHARED((NS * G, D), jnp.float32),
            [pltpu.SemaphoreType.DMA] * NSLOT,
            [pltpu.SemaphoreType.DMA] * NSLOT,
        ],
    )
    def body(idx_hbm, msg_hbm, out_hbm, idx_v, buf_v, lin_v, acc_s,
             gsem, ssem):
        ss = lax.axis_index("s")
        cc = lax.axis_index("c")
        wid = cc * NS + ss
        obase = wid * CHUNK0
        abase = ss * G
        for t in range(G // L):
            lin_v[0, pl.ds(t * L, L)] = (
                jnp.arange(L, dtype=jnp.int32) + abase + t * L
            )
        pltpu.sync_copy(idx_hbm.at[wid], idx_v)

        def g_start(j, p, slot):
            pltpu.async_copy(
                msg_hbm.at[idx_v.at[j, p]], buf_v.at[slot], gsem[slot]
            )

        def g_wait(slot):
            pltpu.make_async_copy(
                msg_hbm.at[idx_v.at[0, 0]], buf_v.at[slot], gsem[slot]
            ).wait()

        def s_start(slot):
            pltpu.async_copy(
                buf_v.at[slot], acc_s.at[lin_v.at[0]], ssem[slot], add=True
            )

        def s_wait(slot):
            pltpu.make_async_copy(
                buf_v.at[slot], acc_s.at[lin_v.at[0]], ssem[slot]
            ).wait()

        def run_pass(p, _):
            for b in range(NSLOT):
                g_start(b, p, b)
            g_wait(0)
            pltpu.async_copy(buf_v.at[0], acc_s.at[pl.ds(abase, G)], ssem[0])
            pltpu.make_async_copy(
                buf_v.at[0], acc_s.at[pl.ds(abase, G)], ssem[0]
            ).wait()
            g_start(NSLOT, p, 0)
            for b in range(1, NSLOT):
                g_wait(b)
                s_start(b)
                s_wait(b)
                g_start(b + NSLOT, p, b)

            def group_step(jj, _):
                for b in range(NSLOT):
                    j = NSLOT * jj + b
                    g_wait(b)
                    s_start(b)
                    s_wait(b)

                    @pl.when(j + NSLOT < NB)
                    def _():
                        g_start(j + NSLOT, p, b)

                return 0

            lax.fori_loop(1, NB // NSLOT, group_step, 0)
            pltpu.sync_copy(
                acc_s.at[pl.ds(abase, G)],
                out_hbm.at[pl.ds(obase + p * G, G)],
            )
            return 0

        lax.fori_loop(0, NP0, run_pass, 0)


    return body(idx_r, msg)


def _tc_matmul_relu(x, wt):
    bm = 1024

    def body(x_ref, w_ref, o_ref):
        o_ref[...] = jnp.maximum(
            jnp.dot(x_ref[...], w_ref[...], preferred_element_type=jnp.float32),
            0.0,
        )

    return pl.pallas_call(
        body,
        grid=(NPAD // bm,),
        in_specs=[
            pl.BlockSpec((bm, D), lambda i: (i, 0)),
            pl.BlockSpec((D, H), lambda i: (0, 0)),
        ],
        out_specs=pl.BlockSpec((bm, H), lambda i: (i, 0)),
        out_shape=jax.ShapeDtypeStruct((NPAD, H), jnp.float32),
    )(x, wt)


def kernel(init_messages, init_attached_features, a2nei, a2attached, W_i, W_h):
    del init_attached_features, a2attached
    fill = (
        jnp.arange(NPAD - N, dtype=jnp.int32)[:, None] * NB
        + jnp.arange(NB, dtype=jnp.int32)[None, :]
    ) % N
    idx = jnp.concatenate([a2nei.astype(jnp.int32), fill], axis=0)
    idx_t = idx.T
    idx_r = (
        idx_t.reshape(NB, NW, NP0, G).transpose(1, 0, 2, 3)
    )

    x = jnp.pad(init_messages, ((0, NPAD - N), (0, 0)))
    msg = _tc_matmul_relu(x, W_i.T)
    for _ in range(DEPTH - 1):
        s = _sc_gather_sum(idx_r, msg)
        msg = _tc_matmul_relu(s, W_h.T)
    return msg[:N]
